# baseline (speedup 1.0000x reference)
import jax
import jax.numpy as jnp
from jax.experimental import pallas as pl
from jax.experimental.pallas import tpu as pltpu

D_MODEL = 4096
NUM_EXPERTS = 64
TOP_K = 8
Z_LOSS = 0.001

TM = 2048
KS = 4
DK = D_MODEL // KS


def _router_kernel(x_ref, w_ref, idx_ref, w_out_ref, counts_ref, scal_ref,
                   acc_ref):
    step = pl.program_id(0)
    kstep = pl.program_id(1)
    nsteps = pl.num_programs(0)

    @pl.when((step == 0) & (kstep == 0))
    def _init():
        counts_ref[...] = jnp.zeros_like(counts_ref)
        scal_ref[0] = 0.0
        scal_ref[1] = jnp.inf
        scal_ref[2] = 0.0
        scal_ref[3] = 0.0
        scal_ref[4] = 0.0
        scal_ref[5] = 0.0
        scal_ref[6] = 0.0
        scal_ref[7] = 0.0

    partial = jax.lax.dot_general(
        w_ref[...], x_ref[...], (((1,), (1,)), ((), ())),
        preferred_element_type=jnp.float32)

    @pl.when(kstep == 0)
    def _acc0():
        acc_ref[...] = partial

    @pl.when(kstep > 0)
    def _accn():
        acc_ref[...] += partial

    @pl.when(kstep == KS - 1)
    def _post():
        logits = acc_ref[...]
        m = jnp.max(logits, axis=0, keepdims=True)
        lm = logits - m
        e = jnp.exp(lm)
        s = jnp.sum(e, axis=0, keepdims=True)
        logs = jnp.log(s)

        ent = logs - jnp.sum(e * lm, axis=0, keepdims=True) / s
        scal_ref[0] += jnp.sum(ent)
        scal_ref[1] = jnp.minimum(scal_ref[1], jnp.min(ent))
        z = m + logs
        scal_ref[2] += jnp.sum(z * z)
        scal_ref[3] += jnp.sum(logits * logits)

        fiota = jax.lax.broadcasted_iota(
            jnp.int32, (NUM_EXPERTS, TM), 0).astype(jnp.float32)
        riota = jnp.float32(NUM_EXPERTS - 1) - fiota
        d = e
        val_rows = []
        rk_rows = []
        for k in range(TOP_K):
            mk = jnp.max(d, axis=0, keepdims=True)
            rk = jnp.max(jnp.where(d == mk, riota, -1.0),
                         axis=0, keepdims=True)
            val_rows.append(mk)
            rk_rows.append(rk)
            if k == 0:
                counts_ref[...] += jnp.sum(
                    (fiota == jnp.float32(NUM_EXPERTS - 1) - rk)
                    .astype(jnp.float32), axis=1, keepdims=True)
            if k < TOP_K - 1:
                d = jnp.where(riota == rk, -1.0, d)
        i8t = (jnp.float32(NUM_EXPERTS - 1)
               - jnp.concatenate(rk_rows, axis=0)).astype(jnp.int32)
        w8t = jnp.concatenate(val_rows, axis=0)
        w8t = w8t / (jnp.sum(w8t, axis=0, keepdims=True) + 1e-9)

        idx_ref[...] = jnp.transpose(i8t)
        w_out_ref[...] = jnp.transpose(w8t)
        scal_ref[4] += jnp.sum(w8t[0, :] - w8t[1, :])
        scal_ref[5] += jnp.sum(w8t[0, :])

        @pl.when(step == nsteps - 1)
        def _fin():
            t_total = jnp.float32(nsteps * TM)
            counts = counts_ref[...]
            cmean = jnp.sum(counts) / NUM_EXPERTS
            cstd = jnp.sqrt(jnp.sum((counts - cmean) ** 2) / NUM_EXPERTS)
            scal_ref[6] = cstd / (cmean + 1e-9)
            scal_ref[0] = scal_ref[0] / t_total
            scal_ref[2] = Z_LOSS * scal_ref[2] / t_total
            scal_ref[3] = jnp.sqrt(scal_ref[3] / (t_total * NUM_EXPERTS))
            scal_ref[4] = scal_ref[4] / t_total
            scal_ref[5] = scal_ref[5] / t_total


def kernel(x, W):
    B, S, D = x.shape
    T = B * S
    h = x.reshape(T, D)
    nsteps = T // TM

    idx, w8, counts, scal = pl.pallas_call(
        _router_kernel,
        grid=(nsteps, KS),
        in_specs=[
            pl.BlockSpec((TM, DK), lambda i, k: (i, k)),
            pl.BlockSpec((NUM_EXPERTS, DK), lambda i, k: (0, k)),
        ],
        out_specs=[
            pl.BlockSpec((TM, TOP_K), lambda i, k: (i, 0)),
            pl.BlockSpec((TM, TOP_K), lambda i, k: (i, 0)),
            pl.BlockSpec((NUM_EXPERTS, 1), lambda i, k: (0, 0)),
            pl.BlockSpec(memory_space=pltpu.SMEM),
        ],
        out_shape=[
            jax.ShapeDtypeStruct((T, TOP_K), jnp.int32),
            jax.ShapeDtypeStruct((T, TOP_K), jnp.float32),
            jax.ShapeDtypeStruct((NUM_EXPERTS, 1), jnp.float32),
            jax.ShapeDtypeStruct((8,), jnp.float32),
        ],
        scratch_shapes=[pltpu.VMEM((NUM_EXPERTS, TM), jnp.float32)],
    )(h, W)

    return (
        idx.astype(jnp.int64),
        w8,
        scal[0],
        scal[1],
        scal[6],
        counts.reshape(NUM_EXPERTS),
        scal[2],
        scal[3],
        scal[4],
        scal[5],
    )


# speedup vs baseline: 1.1450x; 1.1450x over previous
import jax
import jax.numpy as jnp
from jax.experimental import pallas as pl
from jax.experimental.pallas import tpu as pltpu

D_MODEL = 4096
NUM_EXPERTS = 64
TOP_K = 8
Z_LOSS = 0.001

TM = 1024


def _router_kernel(x_ref, w_ref, idx_ref, w_out_ref, counts_ref, scal_ref):
    step = pl.program_id(0)
    nsteps = pl.num_programs(0)

    @pl.when(step == 0)
    def _init():
        counts_ref[...] = jnp.zeros_like(counts_ref)
        scal_ref[0] = 0.0
        scal_ref[1] = jnp.inf
        scal_ref[2] = 0.0
        scal_ref[3] = 0.0
        scal_ref[4] = 0.0
        scal_ref[5] = 0.0
        scal_ref[6] = 0.0
        scal_ref[7] = 0.0

    logits = jax.lax.dot_general(
        w_ref[...], x_ref[...], (((1,), (1,)), ((), ())),
        preferred_element_type=jnp.float32)

    m = jnp.max(logits, axis=0, keepdims=True)
    lm = logits - m
    e = jnp.exp(lm)
    s = jnp.sum(e, axis=0, keepdims=True)
    logs = jnp.log(s)

    ent = logs - jnp.sum(e * lm, axis=0, keepdims=True) / s
    scal_ref[0] += jnp.sum(ent)
    scal_ref[1] = jnp.minimum(scal_ref[1], jnp.min(ent))
    z = m + logs
    scal_ref[2] += jnp.sum(z * z)
    scal_ref[3] += jnp.sum(logits * logits)

    fiota = jax.lax.broadcasted_iota(
        jnp.int32, (NUM_EXPERTS, TM), 0).astype(jnp.float32)
    riota = jnp.float32(NUM_EXPERTS - 1) - fiota
    d = e
    val_rows = []
    rk_rows = []
    for k in range(TOP_K):
        mk = jnp.max(d, axis=0, keepdims=True)
        rk = jnp.max(jnp.where(d == mk, riota, -1.0), axis=0, keepdims=True)
        val_rows.append(mk)
        rk_rows.append(rk)
        if k == 0:
            counts_ref[...] += jnp.sum(
                (fiota == jnp.float32(NUM_EXPERTS - 1) - rk).astype(jnp.float32),
                axis=1, keepdims=True)
        if k < TOP_K - 1:
            d = jnp.where(riota == rk, -1.0, d)
    i8t = (jnp.float32(NUM_EXPERTS - 1)
           - jnp.concatenate(rk_rows, axis=0)).astype(jnp.int32)
    w8t = jnp.concatenate(val_rows, axis=0)
    w8t = w8t / (jnp.sum(w8t, axis=0, keepdims=True) + 1e-9)

    idx_ref[...] = jnp.transpose(i8t)
    w_out_ref[...] = jnp.transpose(w8t)
    scal_ref[4] += jnp.sum(w8t[0, :] - w8t[1, :])
    scal_ref[5] += jnp.sum(w8t[0, :])

    @pl.when(step == nsteps - 1)
    def _fin():
        t_total = jnp.float32(nsteps * TM)
        counts = counts_ref[...]
        cmean = jnp.sum(counts) / NUM_EXPERTS
        cstd = jnp.sqrt(jnp.sum((counts - cmean) ** 2) / NUM_EXPERTS)
        scal_ref[6] = cstd / (cmean + 1e-9)
        scal_ref[0] = scal_ref[0] / t_total
        scal_ref[2] = Z_LOSS * scal_ref[2] / t_total
        scal_ref[3] = jnp.sqrt(scal_ref[3] / (t_total * NUM_EXPERTS))
        scal_ref[4] = scal_ref[4] / t_total
        scal_ref[5] = scal_ref[5] / t_total


def kernel(x, W):
    B, S, D = x.shape
    T = B * S
    h = x.reshape(T, D)
    nsteps = T // TM

    idx, w8, counts, scal = pl.pallas_call(
        _router_kernel,
        grid=(nsteps,),
        in_specs=[
            pl.BlockSpec((TM, D), lambda i: (i, 0)),
            pl.BlockSpec((NUM_EXPERTS, D), lambda i: (0, 0)),
        ],
        out_specs=[
            pl.BlockSpec((TM, TOP_K), lambda i: (i, 0)),
            pl.BlockSpec((TM, TOP_K), lambda i: (i, 0)),
            pl.BlockSpec((NUM_EXPERTS, 1), lambda i: (0, 0)),
            pl.BlockSpec(memory_space=pltpu.SMEM),
        ],
        out_shape=[
            jax.ShapeDtypeStruct((T, TOP_K), jnp.int32),
            jax.ShapeDtypeStruct((T, TOP_K), jnp.float32),
            jax.ShapeDtypeStruct((NUM_EXPERTS, 1), jnp.float32),
            jax.ShapeDtypeStruct((8,), jnp.float32),
        ],
    )(h, W)

    return (
        idx.astype(jnp.int64),
        w8,
        scal[0],
        scal[1],
        scal[6],
        counts.reshape(NUM_EXPERTS),
        scal[2],
        scal[3],
        scal[4],
        scal[5],
    )
